# baseline (speedup 1.0000x reference)
import functools

import jax
import jax.numpy as jnp
from jax import lax
from jax.experimental import pallas as pl
from jax.experimental.pallas import tpu as pltpu
from jax.experimental.pallas import tpu_sc as plsc

B = 16384
D = 64
NCTX = 6
NC, NS, L = 2, 16, 16
NW = NC * NS
BPW = B // NW
C = 64
NCHUNK = BPW // C
G = C // L

QT = 250880
BPQ = 1024
NBLK = QT // BPQ
VCOLS = (1000010 + BPQ - 1) // BPQ - 1


def _pack_body(e0, e1, e2, e3, out):
    half = jnp.uint32(0x8000)
    topm = jnp.uint32(0xFFFF0000)
    halves = []
    for ea, eb in ((e0, e1), (e2, e3)):
        au = lax.bitcast_convert_type(jnp.swapaxes(ea[...], 0, 1), jnp.uint32)
        bu = lax.bitcast_convert_type(jnp.swapaxes(eb[...], 0, 1), jnp.uint32)
        w = ((au + half) >> 16) | ((bu + half) & topm)
        halves.append(lax.bitcast_convert_type(w, jnp.float32))
    out[...] = jnp.concatenate(halves, axis=1)


def _tc_pack(emb_t):
    in_specs = [
        pl.BlockSpec((D, BPQ),
                     functools.partial(
                         lambda g, k: (0, jnp.minimum(g + k * NBLK, VCOLS)),
                         k=kk))
        for kk in range(4)
    ]
    return pl.pallas_call(
        _pack_body,
        grid=(NBLK,),
        in_specs=in_specs,
        out_specs=pl.BlockSpec((BPQ, 2 * D), lambda g: (g, 0)),
        out_shape=jax.ShapeDtypeStruct((QT, 2 * D), jnp.float32),
    )(emb_t, emb_t, emb_t, emb_t)


def _log_sigmoid(t):
    a = jnp.minimum(t, 0.0)
    u = jnp.exp(-jnp.abs(t))
    s = u / (u + 2.0)
    p = s * s
    poly = 1.0 + p * (1.0 / 3.0 + p * (1.0 / 5.0 + p * (1.0 / 7.0
                + p * (1.0 / 9.0 + p * (1.0 / 11.0)))))
    return a - 2.0 * s * poly


def _sc_body(x_h, ei_h, eo_h, out_h, x_v, idx_b, kb, cen0, cen1, ctx0, ctx1,
             out_v, sem0, sem1):
    wid = lax.axis_index("s") * NC + lax.axis_index("c")
    base = wid * BPW
    iota = lax.iota(jnp.int32, L)

    pltpu.sync_copy(x_h.at[pl.ds(base, BPW)], x_v)

    def ext_body(g, _):
        rows = g * L + iota
        for c in range(7):
            col = jnp.full((L,), c, jnp.int32)
            t = plsc.load_gather(x_v, [rows, col])
            k = ((t >= QT).astype(jnp.int32)
                 + (t >= 2 * QT).astype(jnp.int32)
                 + (t >= 3 * QT).astype(jnp.int32))
            idx_b[c, pl.ds(g * L, L)] = t - k * QT
            kb[c, pl.ds(g * L, L)] = k
        return 0

    lax.fori_loop(0, BPW // L, ext_body, 0)

    cens = (cen0, cen1)
    ctxs = (ctx0, ctx1)
    sems = (sem0, sem1)

    def fire(kc):
        p = kc % 2
        cps = [pltpu.async_copy(
            ei_h.at[idx_b.at[0, pl.ds(kc * C, C)]], cens[p], sems[p])]
        for j in range(NCTX):
            cps.append(pltpu.async_copy(
                eo_h.at[idx_b.at[1 + j, pl.ds(kc * C, C)]], ctxs[p].at[j],
                sems[p]))
        return cps

    pend = {0: fire(0)}
    for kc in range(NCHUNK):
        if kc + 1 < NCHUNK:
            pend[kc + 1] = fire(kc + 1)
        for cp in pend.pop(kc):
            cp.wait()
        cen = cens[kc % 2]
        ctx = ctxs[kc % 2]

        def g_body(g, _, cen=cen, ctx=ctx, kc=kc):
            rows = g * L + iota

            wbs, hms = [], []
            for c in range(7):
                kv = kb[c, pl.ds(kc * C + g * L, L)]
                wbs.append((kv >> 1) * D)
                hms.append((kv & 1) == 1)

            def d_body(d, accs):
                wv0 = wbs[0] + d
                vc = _sel(plsc.load_gather(cen, [rows, wv0]), hms[0])
                new = []
                for j in range(NCTX):
                    wvj = wbs[1 + j] + d
                    vx = _sel(plsc.load_gather(
                        ctx, [jnp.full((L,), j, jnp.int32), rows, wvj]),
                        hms[1 + j])
                    new.append(accs[j] + vc * vx)
                return tuple(new)

            accs = lax.fori_loop(
                0, D, d_body,
                tuple(jnp.zeros((L,), jnp.float32) for _ in range(NCTX)))
            for j in range(NCTX):
                out_v[j, pl.ds(kc * C + g * L, L)] = _log_sigmoid(-accs[j])
            return 0

        lax.fori_loop(0, G, g_body, 0)

    pltpu.sync_copy(out_v, out_h.at[:, pl.ds(base, BPW)])


def _sel(word_f32, hi_mask):
    w = plsc.bitcast(word_f32, jnp.uint32)
    v = jnp.where(hi_mask, w & jnp.uint32(0xFFFF0000), w << 16)
    return plsc.bitcast(v, jnp.float32)


@functools.partial(jax.jit, static_argnums=())
def _sc_call(x, ei_p, eo_p):
    mesh = plsc.VectorSubcoreMesh(
        core_axis_name="c", subcore_axis_name="s", num_cores=NC,
        num_subcores=NS)
    return pl.kernel(
        _sc_body,
        out_type=jax.ShapeDtypeStruct((NCTX, B), jnp.float32),
        mesh=mesh,
        compiler_params=pltpu.CompilerParams(
            needs_layout_passes=False, use_tc_tiling_on_sc=False),
        scratch_types=[
            pltpu.VMEM((BPW, 7), jnp.int32),
            pltpu.VMEM((7, BPW), jnp.int32),
            pltpu.VMEM((7, BPW), jnp.int32),
            pltpu.VMEM((C, 2 * D), jnp.float32),
            pltpu.VMEM((C, 2 * D), jnp.float32),
            pltpu.VMEM((NCTX, C, 2 * D), jnp.float32),
            pltpu.VMEM((NCTX, C, 2 * D), jnp.float32),
            pltpu.VMEM((NCTX, BPW), jnp.float32),
            pltpu.SemaphoreType.DMA,
            pltpu.SemaphoreType.DMA,
        ],
    )(x, ei_p, eo_p)


def _jnp_pack(emb):
    half = jnp.uint32(0x8000)
    topm = jnp.uint32(0xFFFF0000)
    u = lax.bitcast_convert_type(emb, jnp.uint32)
    u0, u1, u2 = u[:QT], u[QT:2 * QT], u[2 * QT:3 * QT]
    u3 = jnp.pad(u[3 * QT:], ((0, 4 * QT - u.shape[0]), (0, 0)))
    w01 = ((u0 + half) >> 16) | ((u1 + half) & topm)
    w23 = ((u2 + half) >> 16) | ((u3 + half) & topm)
    return lax.bitcast_convert_type(
        jnp.concatenate([w01, w23], axis=1), jnp.float32)


def kernel(x, emb_i, emb_o):
    out = _sc_call(x, _jnp_pack(emb_i), _jnp_pack(emb_o))
    return jnp.swapaxes(out, 0, 1).reshape(B, 1, NCTX)


# speedup vs baseline: 2.9099x; 2.9099x over previous
import functools

import jax
import jax.numpy as jnp
from jax import lax
from jax.experimental import pallas as pl
from jax.experimental.pallas import tpu as pltpu
from jax.experimental.pallas import tpu_sc as plsc

B = 16384
D = 64
NCTX = 6
NC, NS, L = 2, 16, 16
NW = NC * NS
BPW = B // NW
C = 32
NCHUNK = BPW // C
G = C // L

QT = 250880
BPQ = 1024
NBLK = QT // BPQ
VCOLS = (1000010 + BPQ - 1) // BPQ - 1


def _pack_body(e0, e1, e2, e3, out):
    half = jnp.uint32(0x8000)
    topm = jnp.uint32(0xFFFF0000)
    halves = []
    for ea, eb in ((e0, e1), (e2, e3)):
        au = lax.bitcast_convert_type(ea[...], jnp.uint32)
        bu = lax.bitcast_convert_type(eb[...], jnp.uint32)
        w = ((au + half) >> 16) | ((bu + half) & topm)
        halves.append(
            lax.bitcast_convert_type(jnp.swapaxes(w, 0, 1), jnp.float32))
    out[...] = jnp.concatenate(halves, axis=1)


def _tc_pack(emb_t):
    in_specs = [
        pl.BlockSpec((D, BPQ),
                     functools.partial(
                         lambda g, k: (0, jnp.minimum(g + k * NBLK, VCOLS)),
                         k=kk))
        for kk in range(4)
    ]
    return pl.pallas_call(
        _pack_body,
        grid=(NBLK,),
        in_specs=in_specs,
        out_specs=pl.BlockSpec((BPQ, 2 * D), lambda g: (g, 0)),
        out_shape=jax.ShapeDtypeStruct((QT, 2 * D), jnp.float32),
    )(emb_t, emb_t, emb_t, emb_t)


def _log_sigmoid(t):
    a = jnp.minimum(t, 0.0)
    u = jnp.exp(-jnp.abs(t))
    s = u / (u + 2.0)
    p = s * s
    poly = 1.0 + p * (1.0 / 3.0 + p * (1.0 / 5.0 + p * (1.0 / 7.0
                + p * (1.0 / 9.0 + p * (1.0 / 11.0)))))
    return a - 2.0 * s * poly


def _sc_body(x_h, ei_h, eo_h, out_h, x_v, idx_b, kb, cen0, cen1, ctx0, ctx1,
             out_v, sem0, sem1):
    wid = lax.axis_index("s") * NC + lax.axis_index("c")
    base = wid * BPW
    iota = lax.iota(jnp.int32, L)

    pltpu.sync_copy(x_h.at[pl.ds(base, BPW)], x_v)

    def ext_body(g, _):
        rows = g * L + iota
        for c in range(7):
            col = jnp.full((L,), c, jnp.int32)
            t = plsc.load_gather(x_v, [rows, col])
            k = ((t >= QT).astype(jnp.int32)
                 + (t >= 2 * QT).astype(jnp.int32)
                 + (t >= 3 * QT).astype(jnp.int32))
            idx_b[c, pl.ds(g * L, L)] = t - k * QT
            kb[c, pl.ds(g * L, L)] = k
        return 0

    lax.fori_loop(0, BPW // L, ext_body, 0)

    cens = (cen0, cen1)
    ctxs = (ctx0, ctx1)
    sems = (sem0, sem1)

    def copies(kc, p):
        cps = [pltpu.make_async_copy(
            ei_h.at[idx_b.at[0, pl.ds(kc * C, C)]], cens[p], sems[p])]
        for j in range(NCTX):
            cps.append(pltpu.make_async_copy(
                eo_h.at[idx_b.at[1 + j, pl.ds(kc * C, C)]],
                ctxs[p].at[pl.ds(j * C, C)], sems[p]))
        return cps

    def fire(kc, p):
        for cp in copies(kc, p):
            cp.start()

    def drain(kc, p):
        for cp in copies(kc, p):
            cp.wait()

    def compute(kc, p):
        cen = cens[p]
        ctx = ctxs[p]

        def g_body(g, _):
            rows = g * L + iota

            rws, wbs, hms = [], [], []
            for c in range(7):
                kv = kb[c, pl.ds(kc * C + g * L, L)]
                rws.append(rows if c == 0 else (c - 1) * C + rows)
                wbs.append((kv >> 1) * D)
                hms.append((kv & 1) == 1)

            U = 8

            def d_body(du, accs):
                accs = list(accs)
                for uu in range(U):
                    d = du * U + uu
                    vc = _sel(plsc.load_gather(cen, [rws[0], wbs[0] + d]),
                              hms[0])
                    for j in range(NCTX):
                        vx = _sel(plsc.load_gather(
                            ctx, [rws[1 + j], wbs[1 + j] + d]), hms[1 + j])
                        accs[j] = accs[j] + vc * vx
                return tuple(accs)

            accs = lax.fori_loop(
                0, D // U, d_body,
                tuple(jnp.zeros((L,), jnp.float32) for _ in range(NCTX)))
            for j in range(NCTX):
                out_v[j, pl.ds(kc * C + g * L, L)] = _log_sigmoid(-accs[j])
            return 0

        lax.fori_loop(0, G, g_body, 0)

    fire(0, 0)
    fire(1, 1)

    def pair_body(i, _):
        ka = 2 * i
        drain(ka, 0)
        compute(ka, 0)

        @pl.when(ka + 2 < NCHUNK)
        def _():
            fire(ka + 2, 0)

        drain(ka + 1, 1)
        compute(ka + 1, 1)

        @pl.when(ka + 3 < NCHUNK)
        def _():
            fire(ka + 3, 1)

        return 0

    lax.fori_loop(0, NCHUNK // 2, pair_body, 0)

    pltpu.sync_copy(out_v, out_h.at[:, pl.ds(base, BPW)])


def _sel(word_f32, hi_mask):
    w = plsc.bitcast(word_f32, jnp.uint32)
    v = jnp.where(hi_mask, w & jnp.uint32(0xFFFF0000), w << 16)
    return plsc.bitcast(v, jnp.float32)


@functools.partial(jax.jit, static_argnums=())
def _sc_call(x, ei_p, eo_p):
    mesh = plsc.VectorSubcoreMesh(
        core_axis_name="c", subcore_axis_name="s", num_cores=NC,
        num_subcores=NS)
    return pl.kernel(
        _sc_body,
        out_type=jax.ShapeDtypeStruct((NCTX, B), jnp.float32),
        mesh=mesh,
        compiler_params=pltpu.CompilerParams(
            needs_layout_passes=False, use_tc_tiling_on_sc=False),
        scratch_types=[
            pltpu.VMEM((BPW, 7), jnp.int32),
            pltpu.VMEM((7, BPW), jnp.int32),
            pltpu.VMEM((7, BPW), jnp.int32),
            pltpu.VMEM((C, 2 * D), jnp.float32),
            pltpu.VMEM((C, 2 * D), jnp.float32),
            pltpu.VMEM((NCTX * C, 2 * D), jnp.float32),
            pltpu.VMEM((NCTX * C, 2 * D), jnp.float32),
            pltpu.VMEM((NCTX, BPW), jnp.float32),
            pltpu.SemaphoreType.DMA,
            pltpu.SemaphoreType.DMA,
        ],
    )(x, ei_p, eo_p)


def _jnp_pack(emb):
    half = jnp.uint32(0x8000)
    topm = jnp.uint32(0xFFFF0000)
    u = lax.bitcast_convert_type(emb, jnp.uint32)
    u0, u1, u2 = u[:QT], u[QT:2 * QT], u[2 * QT:3 * QT]
    u3 = jnp.pad(u[3 * QT:], ((0, 4 * QT - u.shape[0]), (0, 0)))
    w01 = ((u0 + half) >> 16) | ((u1 + half) & topm)
    w23 = ((u2 + half) >> 16) | ((u3 + half) & topm)
    return lax.bitcast_convert_type(
        jnp.concatenate([w01, w23], axis=1), jnp.float32)


def kernel(x, emb_i, emb_o):
    ei_p = _tc_pack(jnp.swapaxes(emb_i, 0, 1))
    eo_p = _tc_pack(jnp.swapaxes(emb_o, 0, 1))
    out = _sc_call(x, ei_p, eo_p)
    return jnp.swapaxes(out, 0, 1).reshape(B, 1, NCTX)


# speedup vs baseline: 4.2432x; 1.4582x over previous
import functools

import jax
import jax.numpy as jnp
from jax import lax
from jax.experimental import pallas as pl
from jax.experimental.pallas import tpu as pltpu
from jax.experimental.pallas import tpu_sc as plsc

B = 16384
D = 64
NCTX = 6
NC, NS, L = 2, 16, 16
NW = NC * NS
BPW = B // NW
C = 64
NCHUNK = BPW // C
G = C // L

QT = 253952
BPQ = 4096
NBLK = QT // BPQ
VCOLS = (1000010 + BPQ - 1) // BPQ - 1


def _pack_body(e0, e1, e2, e3, out):
    half = jnp.uint32(0x8000)
    topm = jnp.uint32(0xFFFF0000)
    halves = []
    for ea, eb in ((e0, e1), (e2, e3)):
        au = lax.bitcast_convert_type(ea[...], jnp.uint32)
        bu = lax.bitcast_convert_type(eb[...], jnp.uint32)
        w = ((au + half) >> 16) | ((bu + half) & topm)
        halves.append(
            lax.bitcast_convert_type(jnp.swapaxes(w, 0, 1), jnp.float32))
    out[...] = jnp.concatenate(halves, axis=1)


def _tc_pack(emb_t):
    in_specs = [
        pl.BlockSpec((D, BPQ),
                     functools.partial(
                         lambda g, k: (0, jnp.minimum(g + k * NBLK, VCOLS)),
                         k=kk))
        for kk in range(4)
    ]
    return pl.pallas_call(
        _pack_body,
        grid=(NBLK,),
        in_specs=in_specs,
        out_specs=pl.BlockSpec((BPQ, 2 * D), lambda g: (g, 0)),
        out_shape=jax.ShapeDtypeStruct((QT, 2 * D), jnp.float32),
    )(emb_t, emb_t, emb_t, emb_t)


def _log_sigmoid(t):
    a = jnp.minimum(t, 0.0)
    u = jnp.exp(-jnp.abs(t))
    s = u / (u + 2.0)
    p = s * s
    poly = 1.0 + p * (1.0 / 3.0 + p * (1.0 / 5.0 + p * (1.0 / 7.0
                + p * (1.0 / 9.0 + p * (1.0 / 11.0)))))
    return a - 2.0 * s * poly


def _sc_body(x_h, ei_h, eo_h, out_h, x_v, idx_b, kb, cen0, cen1, ctx0, ctx1,
             out_v, sem0, sem1):
    wid = lax.axis_index("s") * NC + lax.axis_index("c")
    base = wid * BPW
    iota = lax.iota(jnp.int32, L)

    pltpu.sync_copy(x_h.at[pl.ds(base, BPW)], x_v)

    def ext_body(g, _):
        rows = g * L + iota
        for c in range(7):
            col = jnp.full((L,), c, jnp.int32)
            t = plsc.load_gather(x_v, [rows, col])
            k = ((t >= QT).astype(jnp.int32)
                 + (t >= 2 * QT).astype(jnp.int32)
                 + (t >= 3 * QT).astype(jnp.int32))
            idx_b[c, pl.ds(g * L, L)] = t - k * QT
            kb[c, pl.ds(g * L, L)] = k
        return 0

    lax.fori_loop(0, BPW // L, ext_body, 0)

    cens = (cen0, cen1)
    ctxs = (ctx0, ctx1)
    sems = (sem0, sem1)

    def copies(kc, p):
        cps = [pltpu.make_async_copy(
            ei_h.at[idx_b.at[0, pl.ds(kc * C, C)]], cens[p], sems[p])]
        for j in range(NCTX):
            cps.append(pltpu.make_async_copy(
                eo_h.at[idx_b.at[1 + j, pl.ds(kc * C, C)]],
                ctxs[p].at[pl.ds(j * C, C)], sems[p]))
        return cps

    def fire(kc, p):
        for cp in copies(kc, p):
            cp.start()

    def drain(kc, p):
        for cp in copies(kc, p):
            cp.wait()

    def compute(kc, p):
        cen = cens[p]
        ctx = ctxs[p]

        def g_body(g, _):
            rows = g * L + iota

            rws, wbs, hms = [], [], []
            for c in range(7):
                kv = kb[c, pl.ds(kc * C + g * L, L)]
                rws.append(rows if c == 0 else (c - 1) * C + rows)
                wbs.append((kv >> 1) * D)
                hms.append((kv & 1) == 1)

            U = 4

            def d_body(du, accs):
                accs = list(accs)
                for uu in range(U):
                    d = du * U + uu
                    vc = _sel(plsc.load_gather(cen, [rws[0], wbs[0] + d]),
                              hms[0])
                    for j in range(NCTX):
                        vx = _sel(plsc.load_gather(
                            ctx, [rws[1 + j], wbs[1 + j] + d]), hms[1 + j])
                        accs[j] = accs[j] + vc * vx
                return tuple(accs)

            accs = lax.fori_loop(
                0, D // U, d_body,
                tuple(jnp.zeros((L,), jnp.float32) for _ in range(NCTX)))
            for j in range(NCTX):
                out_v[j, pl.ds(kc * C + g * L, L)] = _log_sigmoid(-accs[j])
            return 0

        lax.fori_loop(0, G, g_body, 0)

    fire(0, 0)
    fire(1, 1)

    def pair_body(i, _):
        ka = 2 * i
        drain(ka, 0)
        compute(ka, 0)

        @pl.when(ka + 2 < NCHUNK)
        def _():
            fire(ka + 2, 0)

        drain(ka + 1, 1)
        compute(ka + 1, 1)

        @pl.when(ka + 3 < NCHUNK)
        def _():
            fire(ka + 3, 1)

        return 0

    lax.fori_loop(0, NCHUNK // 2, pair_body, 0)

    pltpu.sync_copy(out_v, out_h.at[:, pl.ds(base, BPW)])


def _sel(word_f32, hi_mask):
    w = plsc.bitcast(word_f32, jnp.uint32)
    v = jnp.where(hi_mask, w & jnp.uint32(0xFFFF0000), w << 16)
    return plsc.bitcast(v, jnp.float32)


@functools.partial(jax.jit, static_argnums=())
def _sc_call(x, ei_p, eo_p):
    mesh = plsc.VectorSubcoreMesh(
        core_axis_name="c", subcore_axis_name="s", num_cores=NC,
        num_subcores=NS)
    return pl.kernel(
        _sc_body,
        out_type=jax.ShapeDtypeStruct((NCTX, B), jnp.float32),
        mesh=mesh,
        compiler_params=pltpu.CompilerParams(
            needs_layout_passes=False, use_tc_tiling_on_sc=False),
        scratch_types=[
            pltpu.VMEM((BPW, 7), jnp.int32),
            pltpu.VMEM((7, BPW), jnp.int32),
            pltpu.VMEM((7, BPW), jnp.int32),
            pltpu.VMEM((C, 2 * D), jnp.float32),
            pltpu.VMEM((C, 2 * D), jnp.float32),
            pltpu.VMEM((NCTX * C, 2 * D), jnp.float32),
            pltpu.VMEM((NCTX * C, 2 * D), jnp.float32),
            pltpu.VMEM((NCTX, BPW), jnp.float32),
            pltpu.SemaphoreType.DMA,
            pltpu.SemaphoreType.DMA,
        ],
    )(x, ei_p, eo_p)


def _jnp_pack(emb):
    half = jnp.uint32(0x8000)
    topm = jnp.uint32(0xFFFF0000)
    u = lax.bitcast_convert_type(emb, jnp.uint32)
    u0, u1, u2 = u[:QT], u[QT:2 * QT], u[2 * QT:3 * QT]
    u3 = jnp.pad(u[3 * QT:], ((0, 4 * QT - u.shape[0]), (0, 0)))
    w01 = ((u0 + half) >> 16) | ((u1 + half) & topm)
    w23 = ((u2 + half) >> 16) | ((u3 + half) & topm)
    return lax.bitcast_convert_type(
        jnp.concatenate([w01, w23], axis=1), jnp.float32)


def kernel(x, emb_i, emb_o):
    ei_p = _tc_pack(jnp.swapaxes(emb_i, 0, 1))
    eo_p = _tc_pack(jnp.swapaxes(emb_o, 0, 1))
    out = _sc_call(x, ei_p, eo_p)
    return jnp.swapaxes(out, 0, 1).reshape(B, 1, NCTX)


# speedup vs baseline: 4.5858x; 1.0807x over previous
import functools

import jax
import jax.numpy as jnp
from jax import lax
from jax.experimental import pallas as pl
from jax.experimental.pallas import tpu as pltpu
from jax.experimental.pallas import tpu_sc as plsc

B = 16384
D = 64
NCTX = 6
NC, NS, L = 2, 16, 16
NW = NC * NS
BPW = B // NW
C = 64
NCHUNK = BPW // C
G = C // L

QT = 253952
BPQ = 8192
NBLK = QT // BPQ
VCOLS = (1000010 + BPQ - 1) // BPQ - 1


def _pack_body(e0, e1, e2, e3, out):
    half = jnp.uint32(0x8000)
    topm = jnp.uint32(0xFFFF0000)
    halves = []
    for ea, eb in ((e0, e1), (e2, e3)):
        au = lax.bitcast_convert_type(ea[...], jnp.uint32)
        bu = lax.bitcast_convert_type(eb[...], jnp.uint32)
        w = ((au + half) >> 16) | ((bu + half) & topm)
        halves.append(
            lax.bitcast_convert_type(jnp.swapaxes(w, 0, 1), jnp.float32))
    out[...] = jnp.concatenate(halves, axis=1)


def _tc_pack(emb_t):
    in_specs = [
        pl.BlockSpec((D, BPQ),
                     functools.partial(
                         lambda g, k: (0, jnp.minimum(g + k * NBLK, VCOLS)),
                         k=kk))
        for kk in range(4)
    ]
    return pl.pallas_call(
        _pack_body,
        grid=(NBLK,),
        in_specs=in_specs,
        out_specs=pl.BlockSpec((BPQ, 2 * D), lambda g: (g, 0)),
        out_shape=jax.ShapeDtypeStruct((QT, 2 * D), jnp.float32),
    )(emb_t, emb_t, emb_t, emb_t)


def _log_sigmoid(t):
    a = jnp.minimum(t, 0.0)
    u = jnp.exp(-jnp.abs(t))
    s = u / (u + 2.0)
    p = s * s
    poly = 1.0 + p * (1.0 / 3.0 + p * (1.0 / 5.0 + p * (1.0 / 7.0
                + p * (1.0 / 9.0 + p * (1.0 / 11.0)))))
    return a - 2.0 * s * poly


def _sc_body(x_h, ei_h, eo_h, out_h, x_v, idx_b, kb, cen0, cen1, ctx0, ctx1,
             out_v, sem0, sem1):
    wid = lax.axis_index("s") * NC + lax.axis_index("c")
    base = wid * BPW
    iota = lax.iota(jnp.int32, L)

    pltpu.sync_copy(x_h.at[pl.ds(base, BPW)], x_v)

    def ext_body(g, _):
        rows = g * L + iota
        for c in range(7):
            col = jnp.full((L,), c, jnp.int32)
            t = plsc.load_gather(x_v, [rows, col])
            k = ((t >= QT).astype(jnp.int32)
                 + (t >= 2 * QT).astype(jnp.int32)
                 + (t >= 3 * QT).astype(jnp.int32))
            idx_b[c, pl.ds(g * L, L)] = t - k * QT
            kb[c, pl.ds(g * L, L)] = k
        return 0

    lax.fori_loop(0, BPW // L, ext_body, 0)

    cens = (cen0, cen1)
    ctxs = (ctx0, ctx1)
    sems = (sem0, sem1)

    def copies(kc, p):
        cps = [pltpu.make_async_copy(
            ei_h.at[idx_b.at[0, pl.ds(kc * C, C)]], cens[p], sems[p])]
        for j in range(NCTX):
            cps.append(pltpu.make_async_copy(
                eo_h.at[idx_b.at[1 + j, pl.ds(kc * C, C)]],
                ctxs[p].at[pl.ds(j * C, C)], sems[p]))
        return cps

    def fire(kc, p):
        for cp in copies(kc, p):
            cp.start()

    def drain(kc, p):
        for cp in copies(kc, p):
            cp.wait()

    def compute(kc, p):
        cen = cens[p]
        ctx = ctxs[p]

        def g_body(g, _):
            rows = g * L + iota

            rws, wbs, hms = [], [], []
            for c in range(7):
                kv = kb[c, pl.ds(kc * C + g * L, L)]
                rws.append(rows if c == 0 else (c - 1) * C + rows)
                wbs.append((kv >> 1) * D)
                hms.append((kv & 1) == 1)

            U = 4

            def d_body(du, accs):
                accs = list(accs)
                for uu in range(U):
                    d = du * U + uu
                    vc = _sel(plsc.load_gather(cen, [rws[0], wbs[0] + d]),
                              hms[0])
                    for j in range(NCTX):
                        vx = _sel(plsc.load_gather(
                            ctx, [rws[1 + j], wbs[1 + j] + d]), hms[1 + j])
                        accs[j] = accs[j] + vc * vx
                return tuple(accs)

            accs = lax.fori_loop(
                0, D // U, d_body,
                tuple(jnp.zeros((L,), jnp.float32) for _ in range(NCTX)))
            for j in range(NCTX):
                out_v[j, pl.ds(kc * C + g * L, L)] = _log_sigmoid(-accs[j])
            return 0

        lax.fori_loop(0, G, g_body, 0)

    fire(0, 0)
    fire(1, 1)

    def pair_body(i, _):
        ka = 2 * i
        drain(ka, 0)
        compute(ka, 0)

        @pl.when(ka + 2 < NCHUNK)
        def _():
            fire(ka + 2, 0)

        drain(ka + 1, 1)
        compute(ka + 1, 1)

        @pl.when(ka + 3 < NCHUNK)
        def _():
            fire(ka + 3, 1)

        return 0

    lax.fori_loop(0, NCHUNK // 2, pair_body, 0)

    pltpu.sync_copy(out_v, out_h.at[:, pl.ds(base, BPW)])


def _sel(word_f32, hi_mask):
    w = plsc.bitcast(word_f32, jnp.uint32)
    v = jnp.where(hi_mask, w & jnp.uint32(0xFFFF0000), w << 16)
    return plsc.bitcast(v, jnp.float32)


@functools.partial(jax.jit, static_argnums=())
def _sc_call(x, ei_p, eo_p):
    mesh = plsc.VectorSubcoreMesh(
        core_axis_name="c", subcore_axis_name="s", num_cores=NC,
        num_subcores=NS)
    return pl.kernel(
        _sc_body,
        out_type=jax.ShapeDtypeStruct((NCTX, B), jnp.float32),
        mesh=mesh,
        compiler_params=pltpu.CompilerParams(
            needs_layout_passes=False, use_tc_tiling_on_sc=False),
        scratch_types=[
            pltpu.VMEM((BPW, 7), jnp.int32),
            pltpu.VMEM((7, BPW), jnp.int32),
            pltpu.VMEM((7, BPW), jnp.int32),
            pltpu.VMEM((C, 2 * D), jnp.float32),
            pltpu.VMEM((C, 2 * D), jnp.float32),
            pltpu.VMEM((NCTX * C, 2 * D), jnp.float32),
            pltpu.VMEM((NCTX * C, 2 * D), jnp.float32),
            pltpu.VMEM((NCTX, BPW), jnp.float32),
            pltpu.SemaphoreType.DMA,
            pltpu.SemaphoreType.DMA,
        ],
    )(x, ei_p, eo_p)


def _jnp_pack(emb):
    half = jnp.uint32(0x8000)
    topm = jnp.uint32(0xFFFF0000)
    u = lax.bitcast_convert_type(emb, jnp.uint32)
    u0, u1, u2 = u[:QT], u[QT:2 * QT], u[2 * QT:3 * QT]
    u3 = jnp.pad(u[3 * QT:], ((0, 4 * QT - u.shape[0]), (0, 0)))
    w01 = ((u0 + half) >> 16) | ((u1 + half) & topm)
    w23 = ((u2 + half) >> 16) | ((u3 + half) & topm)
    return lax.bitcast_convert_type(
        jnp.concatenate([w01, w23], axis=1), jnp.float32)


def kernel(x, emb_i, emb_o):
    ei_p = _tc_pack(jnp.swapaxes(emb_i, 0, 1))
    eo_p = _tc_pack(jnp.swapaxes(emb_o, 0, 1))
    out = _sc_call(x, ei_p, eo_p)
    return jnp.swapaxes(out, 0, 1).reshape(B, 1, NCTX)
